# Initial kernel scaffold; baseline (speedup 1.0000x reference)
import jax
import jax.numpy as jnp
from jax.experimental import pallas as pl


def kernel(embedding, w_relation, triplets):
    raise NotImplementedError("write your pallas kernel here")


# speedup vs baseline: 2.1163x; 2.1163x over previous
import functools

import jax
import jax.numpy as jnp
from jax import lax
from jax.experimental import pallas as pl
from jax.experimental.pallas import tpu as pltpu
from jax.experimental.pallas import tpu_sc as plsc

H = 64
C = 128
L = 16
NC = 2
NS = 16
NW = NC * NS
N_REL = 100


@functools.partial(jax.jit, static_argnames=("npw",))
def _sc_score(emb, wrel, src, rel, dst, npw):
    np_total = src.shape[0]
    mesh = plsc.VectorSubcoreMesh(core_axis_name="c", subcore_axis_name="s")

    @functools.partial(
        pl.kernel,
        mesh=mesh,
        compiler_params=pltpu.CompilerParams(
            needs_layout_passes=False, use_tc_tiling_on_sc=False),
        out_type=jax.ShapeDtypeStruct((np_total,), jnp.float32),
        scratch_types=[
            pltpu.VMEM((N_REL, H), jnp.float32),
            pltpu.VMEM((2, C), jnp.int32),
            pltpu.VMEM((2, C), jnp.int32),
            pltpu.VMEM((2, C), jnp.int32),
            pltpu.VMEM((2, C, H), jnp.float32),
            pltpu.VMEM((2, C, H), jnp.float32),
            pltpu.VMEM((2, C), jnp.float32),
            pltpu.SemaphoreType.DMA,
            pltpu.SemaphoreType.DMA,
        ],
    )
    def k(emb_h, wrel_h, src_h, rel_h, dst_h, out_h,
          wrel_v, src_v, dst_v, rel_v, s_v, o_v, out_v, sem0, sem1):
        wid = lax.axis_index("s") * NC + lax.axis_index("c")
        base_chunk = wid * npw

        pltpu.sync_copy(wrel_h, wrel_v)

        def issue(t, b):
            sem = sem0 if b == 0 else sem1
            off = (base_chunk + t) * C
            pltpu.sync_copy(src_h.at[pl.ds(off, C)], src_v.at[b])
            pltpu.sync_copy(dst_h.at[pl.ds(off, C)], dst_v.at[b])
            pltpu.sync_copy(rel_h.at[pl.ds(off, C)], rel_v.at[b])
            pltpu.async_copy(emb_h.at[src_v.at[b]], s_v.at[b], sem)
            pltpu.async_copy(emb_h.at[dst_v.at[b]], o_v.at[b], sem)

        def compute(t, b):
            sem = sem0 if b == 0 else sem1
            pltpu.make_async_copy(emb_h.at[src_v.at[b]], s_v.at[b], sem).wait()
            pltpu.make_async_copy(emb_h.at[dst_v.at[b]], o_v.at[b], sem).wait()

            def iblk(i0, _):
                rows = i0 * L + lax.iota(jnp.int32, L)
                relv = rel_v[b, pl.ds(i0 * L, L)]
                accs = [jnp.zeros((L,), jnp.float32) for _ in range(4)]
                for d in range(H):
                    cols = jnp.full((L,), d, jnp.int32)
                    sd = plsc.load_gather(s_v.at[b], [rows, cols])
                    od = plsc.load_gather(o_v.at[b], [rows, cols])
                    rd = plsc.load_gather(wrel_v, [relv, cols])
                    accs[d % 4] = accs[d % 4] + sd * od * rd
                out_v[b, pl.ds(i0 * L, L)] = (
                    (accs[0] + accs[1]) + (accs[2] + accs[3]))
                return _

            lax.fori_loop(0, C // L, iblk, None)
            off = (base_chunk + t) * C
            pltpu.sync_copy(out_v.at[b], out_h.at[pl.ds(off, C)])

        issue(0, 0)

        def outer(g, _):
            t0 = g * 2
            issue(t0 + 1, 1)
            compute(t0, 0)

            @pl.when(t0 + 2 < npw)
            def _issue_next():
                issue(t0 + 2, 0)

            compute(t0 + 1, 1)
            return _

        lax.fori_loop(0, npw // 2, outer, None)

    return k(emb, wrel, src, rel, dst)


def kernel(embedding, w_relation, triplets):
    n = triplets.shape[0]
    n_chunks = -(-n // C)
    npw = -(-n_chunks // NW)
    np_total = NW * npw * C
    pad = np_total - n
    src = jnp.pad(triplets[:, 0].astype(jnp.int32), (0, pad))
    rel = jnp.pad(triplets[:, 1].astype(jnp.int32), (0, pad))
    dst = jnp.pad(triplets[:, 2].astype(jnp.int32), (0, pad))
    out = _sc_score(embedding, w_relation.astype(jnp.float32),
                    src, rel, dst, npw)
    return out[:n]


# speedup vs baseline: 2.2909x; 1.0825x over previous
import functools

import jax
import jax.numpy as jnp
from jax import lax
from jax.experimental import pallas as pl
from jax.experimental.pallas import tpu as pltpu
from jax.experimental.pallas import tpu_sc as plsc

H = 64
C = 128
L = 16
NC = 2
NS = 16
NW = NC * NS
N_REL = 100


@functools.partial(jax.jit, static_argnames=("npw",))
def _sc_score(emb, wrel, idx_all, npw):
    n_chunks = idx_all.shape[0]
    np_total = n_chunks * C
    mesh = plsc.VectorSubcoreMesh(core_axis_name="c", subcore_axis_name="s")

    @functools.partial(
        pl.kernel,
        mesh=mesh,
        compiler_params=pltpu.CompilerParams(
            needs_layout_passes=False, use_tc_tiling_on_sc=False),
        out_type=jax.ShapeDtypeStruct((np_total,), jnp.float32),
        scratch_types=[
            pltpu.VMEM((N_REL, H), jnp.float32),
            pltpu.VMEM((2, 3, C), jnp.int32),
            pltpu.VMEM((2, C, H), jnp.float32),
            pltpu.VMEM((2, C, H), jnp.float32),
            pltpu.VMEM((2, C), jnp.float32),
            pltpu.SemaphoreType.DMA,
            pltpu.SemaphoreType.DMA,
            pltpu.SemaphoreType.DMA,
            pltpu.SemaphoreType.DMA,
            pltpu.SemaphoreType.DMA,
            pltpu.SemaphoreType.DMA,
        ],
    )
    def k(emb_h, wrel_h, idx_h, out_h,
          wrel_v, idx_v, s_v, o_v, out_v,
          semi0, semi1, semr0, semr1, semo0, semo1):
        wid = lax.axis_index("s") * NC + lax.axis_index("c")
        base_chunk = wid * npw
        semi = (semi0, semi1)
        semr = (semr0, semr1)
        semo = (semo0, semo1)

        pltpu.sync_copy(wrel_h, wrel_v)

        def issue_idx(t, b):
            pltpu.async_copy(idx_h.at[base_chunk + t], idx_v.at[b], semi[b])

        def wait_idx(b):
            pltpu.make_async_copy(idx_h.at[0], idx_v.at[b], semi[b]).wait()

        def issue_rows(b):
            pltpu.async_copy(emb_h.at[idx_v.at[b, 0]], s_v.at[b], semr[b])
            pltpu.async_copy(emb_h.at[idx_v.at[b, 2]], o_v.at[b], semr[b])

        def wait_rows(b):
            pltpu.make_async_copy(emb_h.at[idx_v.at[b, 0]], s_v.at[b],
                                  semr[b]).wait()
            pltpu.make_async_copy(emb_h.at[idx_v.at[b, 2]], o_v.at[b],
                                  semr[b]).wait()

        def store_out(t, b):
            off = (base_chunk + t) * C
            pltpu.async_copy(out_v.at[b], out_h.at[pl.ds(off, C)], semo[b])

        def wait_out(t, b):
            off = (base_chunk + t) * C
            pltpu.make_async_copy(out_v.at[b], out_h.at[pl.ds(off, C)],
                                  semo[b]).wait()

        def compute(b):
            def iblk(i0, _):
                rows = i0 * L + lax.iota(jnp.int32, L)
                relv = idx_v[b, 1, pl.ds(i0 * L, L)]
                accs = [jnp.zeros((L,), jnp.float32) for _ in range(4)]
                for d in range(H):
                    cols = jnp.full((L,), d, jnp.int32)
                    sd = plsc.load_gather(s_v.at[b], [rows, cols])
                    od = plsc.load_gather(o_v.at[b], [rows, cols])
                    rd = plsc.load_gather(wrel_v, [relv, cols])
                    accs[d % 4] = accs[d % 4] + sd * od * rd
                out_v[b, pl.ds(i0 * L, L)] = (
                    (accs[0] + accs[1]) + (accs[2] + accs[3]))
                return _

            lax.fori_loop(0, C // L, iblk, None)

        issue_idx(0, 0)
        issue_idx(1, 1)
        wait_idx(0)
        issue_rows(0)

        def step(t, b):
            wait_rows(b)

            @pl.when(t + 1 < npw)
            def _():
                wait_idx(1 - b)
                issue_rows(1 - b)

            @pl.when(t >= 2)
            def _():
                wait_out(t - 2, b)

            compute(b)
            @pl.when(t + 2 < npw)
            def _():
                issue_idx(t + 2, b)

            store_out(t, b)

        def outer(g, _):
            step(g * 2, 0)
            step(g * 2 + 1, 1)
            return _

        lax.fori_loop(0, npw // 2, outer, None)
        wait_out(npw - 2, 0)
        wait_out(npw - 1, 1)

    return k(emb, wrel, idx_all)


def kernel(embedding, w_relation, triplets):
    n = triplets.shape[0]
    n_chunks = -(-n // C)
    npw = -(-n_chunks // NW)
    n_chunks = NW * npw
    np_total = n_chunks * C
    trip = jnp.pad(triplets.astype(jnp.int32), ((0, np_total - n), (0, 0)))
    idx_all = trip.reshape(n_chunks, C, 3).transpose(0, 2, 1)
    out = _sc_score(embedding, w_relation.astype(jnp.float32), idx_all, npw)
    return out[:n]


# speedup vs baseline: 9.0225x; 3.9384x over previous
import functools

import jax
import jax.numpy as jnp
from jax import lax
from jax.experimental import pallas as pl
from jax.experimental.pallas import tpu as pltpu
from jax.experimental.pallas import tpu_sc as plsc

H = 64
C = 128
L = 16
NC = 2
NS = 16
NW = NC * NS
N_REL = 100


@functools.partial(jax.jit, static_argnames=("npw",))
def _sc_score(emb, wrel, idx_all, npw):
    n_chunks = idx_all.shape[0]
    np_total = n_chunks * C
    mesh = plsc.VectorSubcoreMesh(core_axis_name="c", subcore_axis_name="s")

    @functools.partial(
        pl.kernel,
        mesh=mesh,
        compiler_params=pltpu.CompilerParams(
            needs_layout_passes=False, use_tc_tiling_on_sc=False),
        out_type=jax.ShapeDtypeStruct((np_total,), jnp.float32),
        scratch_types=[
            pltpu.VMEM((N_REL, H), jnp.float32),
            pltpu.VMEM((2, 3, C), jnp.int32),
            pltpu.VMEM((2, C, H), jnp.float32),
            pltpu.VMEM((2, C, H), jnp.float32),
            pltpu.VMEM((2, C), jnp.float32),
            pltpu.SemaphoreType.DMA,
            pltpu.SemaphoreType.DMA,
            pltpu.SemaphoreType.DMA,
            pltpu.SemaphoreType.DMA,
            pltpu.SemaphoreType.DMA,
            pltpu.SemaphoreType.DMA,
        ],
    )
    def k(emb_h, wrel_h, idx_h, out_h,
          wrel_v, idx_v, s_v, o_v, out_v,
          semi0, semi1, semr0, semr1, semo0, semo1):
        wid = lax.axis_index("s") * NC + lax.axis_index("c")
        base_chunk = wid * npw
        semi = (semi0, semi1)
        semr = (semr0, semr1)
        semo = (semo0, semo1)

        pltpu.sync_copy(wrel_h, wrel_v)

        def issue_idx(t, b):
            pltpu.async_copy(idx_h.at[base_chunk + t], idx_v.at[b], semi[b])

        def wait_idx(b):
            pltpu.make_async_copy(idx_h.at[0], idx_v.at[b], semi[b]).wait()

        def issue_rows(b):
            pltpu.async_copy(emb_h.at[idx_v.at[b, 0]], s_v.at[b], semr[b])
            pltpu.async_copy(emb_h.at[idx_v.at[b, 2]], o_v.at[b], semr[b])

        def wait_rows(b):
            pltpu.make_async_copy(emb_h.at[idx_v.at[b, 0]], s_v.at[b],
                                  semr[b]).wait()
            pltpu.make_async_copy(emb_h.at[idx_v.at[b, 2]], o_v.at[b],
                                  semr[b]).wait()

        def store_out(t, b):
            off = (base_chunk + t) * C
            pltpu.async_copy(out_v.at[b], out_h.at[pl.ds(off, C)], semo[b])

        def wait_out(t, b):
            off = (base_chunk + t) * C
            pltpu.make_async_copy(out_v.at[b], out_h.at[pl.ds(off, C)],
                                  semo[b]).wait()

        def compute(b):
            def iblk(i0, _):
                rows = i0 * L + lax.iota(jnp.int32, L)
                relv = idx_v[b, 1, pl.ds(i0 * L, L)]
                lane = lax.iota(jnp.int32, L)
                accs = [jnp.zeros((L,), jnp.float32) for _ in range(4)]
                for d in range(H):
                    cols = (lane + d) & (H - 1)
                    sd = plsc.load_gather(s_v.at[b], [rows, cols])
                    od = plsc.load_gather(o_v.at[b], [rows, cols])
                    rd = plsc.load_gather(wrel_v, [relv, cols])
                    accs[d % 4] = accs[d % 4] + sd * od * rd
                out_v[b, pl.ds(i0 * L, L)] = (
                    (accs[0] + accs[1]) + (accs[2] + accs[3]))
                return _

            lax.fori_loop(0, C // L, iblk, None)

        issue_idx(0, 0)
        issue_idx(1, 1)
        wait_idx(0)
        issue_rows(0)

        def step(t, b):
            wait_rows(b)

            @pl.when(t + 1 < npw)
            def _():
                wait_idx(1 - b)
                issue_rows(1 - b)

            @pl.when(t >= 2)
            def _():
                wait_out(t - 2, b)

            compute(b)
            @pl.when(t + 2 < npw)
            def _():
                issue_idx(t + 2, b)

            store_out(t, b)

        def outer(g, _):
            step(g * 2, 0)
            step(g * 2 + 1, 1)
            return _

        lax.fori_loop(0, npw // 2, outer, None)
        wait_out(npw - 2, 0)
        wait_out(npw - 1, 1)

    return k(emb, wrel, idx_all)


def kernel(embedding, w_relation, triplets):
    n = triplets.shape[0]
    n_chunks = -(-n // C)
    npw = -(-n_chunks // NW)
    n_chunks = NW * npw
    np_total = n_chunks * C
    trip = jnp.pad(triplets.astype(jnp.int32), ((0, np_total - n), (0, 0)))
    idx_all = trip.reshape(n_chunks, C, 3).transpose(0, 2, 1)
    out = _sc_score(embedding, w_relation.astype(jnp.float32), idx_all, npw)
    return out[:n]
